# baseline (speedup 1.0000x reference)
import functools

import jax
import jax.numpy as jnp
from jax import lax
from jax.experimental import pallas as pl
from jax.experimental.pallas import tpu as pltpu
from jax.experimental.pallas import tpu_sc as plsc

_B, _T, _E = 4096, 200, 32
_N = _B * _T
_NW = 32
_PW = _N // _NW
_GS = 1280
_G = _PW // _GS

_mesh = plsc.VectorSubcoreMesh(core_axis_name="c", subcore_axis_name="s")


@functools.partial(
    pl.kernel,
    out_type=jax.ShapeDtypeStruct((_N, _E), jnp.float32),
    mesh=_mesh,
    scratch_types=[
        pltpu.VMEM((_PW,), jnp.int32),
        pltpu.VMEM((2, _GS, _E), jnp.float32),
        pltpu.SemaphoreType.DMA,
        pltpu.SemaphoreType.DMA,
        pltpu.SemaphoreType.DMA,
        pltpu.SemaphoreType.DMA,
    ],
    compiler_params=pltpu.CompilerParams(use_tc_tiling_on_sc=False),
)
def _sc_gather(idx_hbm, table_hbm, out_hbm, idx_v, buf_v, g0, g1, o0, o1):
    wid = lax.axis_index("s") * 2 + lax.axis_index("c")
    base = wid * _PW
    pltpu.sync_copy(idx_hbm.at[pl.ds(base, _PW)], idx_v)

    def start_gather(g, slot, sem):
        pltpu.async_copy(
            table_hbm.at[idx_v.at[pl.ds(g * _GS, _GS)]], buf_v.at[slot], sem
        )

    def wait_gather(slot, sem):
        pltpu.make_async_copy(
            table_hbm.at[idx_v.at[pl.ds(0, _GS)]], buf_v.at[slot], sem
        ).wait()

    start_gather(0, 0, g0)

    _G2 = _G // 2

    @pl.loop(0, _G2)
    def _pair(i):
        ga, gb = 2 * i, 2 * i + 1
        start_gather(gb, 1, g1)
        wait_gather(0, g0)
        sa = pltpu.async_copy(
            buf_v.at[0], out_hbm.at[pl.ds(base + ga * _GS, _GS)], o0
        )
        sa.wait()

        @pl.when(i < _G2 - 1)
        def _():
            start_gather(gb + 1, 0, g0)

        wait_gather(1, g1)
        sb = pltpu.async_copy(
            buf_v.at[1], out_hbm.at[pl.ds(base + gb * _GS, _GS)], o1
        )
        sb.wait()


def kernel(input_, table):
    idx = input_.reshape(_N)
    out = _sc_gather(idx, table)
    return out.reshape(_B, _T, _E)


# speedup vs baseline: 1.0019x; 1.0019x over previous
import functools

import jax
import jax.numpy as jnp
from jax import lax
from jax.experimental import pallas as pl
from jax.experimental.pallas import tpu as pltpu
from jax.experimental.pallas import tpu_sc as plsc

_B, _T, _E = 4096, 200, 32
_N = _B * _T
_NW = 32
_PW = _N // _NW
_BW = _B // _NW
_NB = 4
_GS = _NB * _T
_G = _BW // _NB

_mesh = plsc.VectorSubcoreMesh(core_axis_name="c", subcore_axis_name="s")


@functools.partial(
    pl.kernel,
    out_type=jax.ShapeDtypeStruct((_B, _T, _E), jnp.float32),
    mesh=_mesh,
    scratch_types=[
        pltpu.VMEM((_PW,), jnp.int32),
        pltpu.VMEM((2, _GS, _E), jnp.float32),
        pltpu.SemaphoreType.DMA,
        pltpu.SemaphoreType.DMA,
        pltpu.SemaphoreType.DMA,
        pltpu.SemaphoreType.DMA,
    ],
    compiler_params=pltpu.CompilerParams(use_tc_tiling_on_sc=False),
)
def _sc_gather(idx_hbm, table_hbm, out_hbm, idx_v, buf_v, g0, g1, o0, o1):
    wid = lax.axis_index("s") * 2 + lax.axis_index("c")
    base = wid * _PW
    bbase = wid * _BW
    pltpu.sync_copy(idx_hbm.at[pl.ds(base, _PW)], idx_v)

    def start_gather(g, slot, sem):
        pltpu.async_copy(
            table_hbm.at[idx_v.at[pl.ds(g * _GS, _GS)]], buf_v.at[slot], sem
        )

    def wait_gather(slot, sem):
        pltpu.make_async_copy(
            table_hbm.at[idx_v.at[pl.ds(0, _GS)]], buf_v.at[slot], sem
        ).wait()

    def store(g, slot, sem):
        for b in range(_NB):
            pltpu.async_copy(
                buf_v.at[slot, pl.ds(b * _T, _T)],
                out_hbm.at[bbase + g * _NB + b],
                sem,
            )

    def wait_store(slot, sem):
        for b in range(_NB):
            pltpu.make_async_copy(
                buf_v.at[slot, pl.ds(b * _T, _T)], out_hbm.at[0], sem
            ).wait()

    start_gather(0, 0, g0)

    _G2 = _G // 2

    @pl.loop(0, _G2)
    def _pair(i):
        ga, gb = 2 * i, 2 * i + 1
        start_gather(gb, 1, g1)
        wait_gather(0, g0)
        store(ga, 0, o0)
        wait_store(0, o0)

        @pl.when(i < _G2 - 1)
        def _():
            start_gather(gb + 1, 0, g0)

        wait_gather(1, g1)
        store(gb, 1, o1)
        wait_store(1, o1)


def kernel(input_, table):
    idx = input_.reshape(_N)
    return _sc_gather(idx, table)
